# baseline (speedup 1.0000x reference)
import functools

import jax
import jax.numpy as jnp
from jax import lax
from jax.experimental import pallas as pl
from jax.experimental.pallas import tpu as pltpu
from jax.experimental.pallas import tpu_sc as plsc

N_TOK = 8192
DIM_IN = 768
DIM_H = 32
MEM_SIZE = 8192
MEM_RATE = 0.5

BLK = 512
CHUNK = 2048
NBLK = N_TOK // BLK


def _sc_geometry():
    try:
        info = plsc.get_sparse_core_info()
        return info.num_cores, info.num_subcores
    except Exception:
        return 2, 16


def _proj_body(z_ref, w_ref, b_ref, out_ref):
    out_ref[...] = (
        lax.dot_general(z_ref[...], w_ref[...], (((1,), (0,)), ((), ())))
        + b_ref[...]
    )


def _argmin_body(proj_ref, mem_ref, psum_ref, msum_ref, idx_ref):
    proj = proj_ref[...]
    prow = psum_ref[...]
    rowid = lax.broadcasted_iota(jnp.int32, (CHUNK, BLK), 0).astype(jnp.float32)

    def body(j, carry):
        minval, minidx = carry
        mem = mem_ref[pl.ds(j * CHUNK, CHUNK), :] * -2.0
        mcol = msum_ref[pl.ds(j * CHUNK, CHUNK), :]
        cross = lax.dot_general(mem, proj, (((1,), (1,)), ((), ())))
        d2 = (prow + mcol) + cross
        d2c = jnp.maximum(d2, 0.0)
        dist = jnp.where(d2c > 0.0, d2c * lax.rsqrt(d2c), 0.0)
        cmin = jnp.min(dist, axis=0, keepdims=True)
        cidx = jnp.min(
            jnp.where(dist == cmin, rowid, jnp.float32(2**24)),
            axis=0, keepdims=True,
        ).astype(jnp.int32) + j * CHUNK
        take = cmin < minval
        newval = jnp.where(take, cmin, minval)
        newval = newval.astype(jnp.bfloat16).astype(jnp.float32)
        return (newval, jnp.where(take, cidx, minidx))

    init = (
        jnp.full((1, BLK), jnp.inf, jnp.float32),
        jnp.zeros((1, BLK), jnp.int32),
    )
    _, minidx = lax.fori_loop(0, MEM_SIZE // CHUNK, body, init)
    idx_ref[...] = minidx.reshape(1, 1, BLK)


def _update_body(proj_ref, closest_ref, w_ref, b_ref, out_ref):
    diff = proj_ref[...] - closest_ref[...]
    out_ref[...] = (
        lax.dot_general(diff, w_ref[...], (((1,), (0,)), ((), ())))
        + b_ref[...]
    ) * MEM_RATE


def _tc_proj(z, w, b2):
    return pl.pallas_call(
        _proj_body,
        grid=(NBLK,),
        in_specs=[
            pl.BlockSpec((BLK, DIM_IN), lambda i: (i, 0)),
            pl.BlockSpec((DIM_IN, DIM_H), lambda i: (0, 0)),
            pl.BlockSpec((1, DIM_H), lambda i: (0, 0)),
        ],
        out_specs=pl.BlockSpec((BLK, DIM_H), lambda i: (i, 0)),
        out_shape=jax.ShapeDtypeStruct((N_TOK, DIM_H), jnp.float32),
    )(z, w, b2)


def _tc_argmin(proj, memory, psum_row, msum_col):
    n_tok = proj.shape[0]
    nblk = n_tok // BLK
    idx3 = pl.pallas_call(
        _argmin_body,
        grid=(nblk,),
        in_specs=[
            pl.BlockSpec((BLK, DIM_H), lambda i: (i, 0)),
            pl.BlockSpec((MEM_SIZE, DIM_H), lambda i: (0, 0)),
            pl.BlockSpec((1, BLK), lambda i: (0, i)),
            pl.BlockSpec((MEM_SIZE, 1), lambda i: (0, 0)),
        ],
        out_specs=pl.BlockSpec((1, 1, BLK), lambda i: (i, 0, 0)),
        out_shape=jax.ShapeDtypeStruct((nblk, 1, BLK), jnp.int32),
    )(proj, memory, psum_row, msum_col)
    return idx3.reshape(n_tok)


def _tc_update(proj, closest, w_upd, b2):
    return pl.pallas_call(
        _update_body,
        grid=(proj.shape[0] // BLK,),
        in_specs=[
            pl.BlockSpec((BLK, DIM_H), lambda i: (i, 0)),
            pl.BlockSpec((BLK, DIM_H), lambda i: (i, 0)),
            pl.BlockSpec((DIM_H, DIM_H), lambda i: (0, 0)),
            pl.BlockSpec((1, DIM_H), lambda i: (0, 0)),
        ],
        out_specs=pl.BlockSpec((BLK, DIM_H), lambda i: (i, 0)),
        out_shape=jax.ShapeDtypeStruct((proj.shape[0], DIM_H), jnp.float32),
    )(proj, closest, w_upd, b2)


def _sc_gather(memory, idx):
    nc, ns = _sc_geometry()
    nw = nc * ns
    n_tok = idx.shape[0]
    bpw = n_tok // nw
    mesh = plsc.VectorSubcoreMesh(core_axis_name="c", subcore_axis_name="s")

    @functools.partial(
        pl.kernel,
        mesh=mesh,
        out_type=jax.ShapeDtypeStruct((n_tok, DIM_H), jnp.float32),
        scratch_types=[
            pltpu.VMEM((bpw,), jnp.int32),
            pltpu.VMEM((bpw, DIM_H), jnp.float32),
            pltpu.SemaphoreType.DMA,
        ],
        compiler_params=pltpu.CompilerParams(use_tc_tiling_on_sc=False),
    )
    def k(table_hbm, idx_hbm, out_hbm, idx_v, rows_v, sem):
        wid = lax.axis_index("s") * nc + lax.axis_index("c")
        base = wid * bpw
        pltpu.sync_copy(idx_hbm.at[pl.ds(base, bpw)], idx_v)
        pltpu.async_copy(table_hbm.at[idx_v], rows_v, sem).wait()
        pltpu.sync_copy(rows_v, out_hbm.at[pl.ds(base, bpw)])

    return k(memory, idx)


def _sc_scatter_add(memory, idx, upd):
    nc, ns = _sc_geometry()
    tok_pw = N_TOK // ns
    rows_init = MEM_SIZE // ns
    rows_out = MEM_SIZE // (nc * ns)
    mesh = plsc.VectorSubcoreMesh(core_axis_name="c", subcore_axis_name="s")

    @functools.partial(
        pl.kernel,
        mesh=mesh,
        out_type=jax.ShapeDtypeStruct((MEM_SIZE, DIM_H), jnp.float32),
        scratch_types=[
            pltpu.VMEM((tok_pw,), jnp.int32),
            pltpu.VMEM((tok_pw, DIM_H), jnp.float32),
            pltpu.VMEM_SHARED((MEM_SIZE, DIM_H), jnp.float32),
        ],
        compiler_params=pltpu.CompilerParams(use_tc_tiling_on_sc=False),
    )
    def k(mem_hbm, idx_hbm, upd_hbm, out_hbm, idx_v, upd_v, shared):
        cid = lax.axis_index("c")
        sid = lax.axis_index("s")
        pltpu.sync_copy(
            mem_hbm.at[pl.ds(sid * rows_init, rows_init)],
            shared.at[pl.ds(sid * rows_init, rows_init)],
        )
        plsc.subcore_barrier()
        base = sid * tok_pw
        pltpu.sync_copy(idx_hbm.at[pl.ds(base, tok_pw)], idx_v)
        pltpu.sync_copy(upd_hbm.at[pl.ds(base, tok_pw)], upd_v)
        pltpu.sync_copy(upd_v, shared.at[idx_v], add=True)
        plsc.subcore_barrier()
        wbase = cid * (MEM_SIZE // nc) + sid * rows_out
        pltpu.sync_copy(
            shared.at[pl.ds(wbase, rows_out)],
            out_hbm.at[pl.ds(wbase, rows_out)],
        )

    return k(memory, idx, upd)


def kernel(visual_embeddings, memory, W_proj, b_proj, W_upd, b_upd):
    b_proj2 = b_proj.reshape(1, DIM_H)
    b_upd2 = b_upd.reshape(1, DIM_H)

    proj = _tc_proj(visual_embeddings, W_proj, b_proj2)
    psum_row = jnp.sum(proj * proj, axis=1)[None, :]
    msum_col = jnp.sum(memory * memory, axis=1)[:, None]

    h = N_TOK // 2
    idx1 = _tc_argmin(proj[:h], memory, psum_row[:, :h], msum_col)
    closest1 = _sc_gather(memory, idx1)
    idx2 = _tc_argmin(proj[h:], memory, psum_row[:, h:], msum_col)
    upd1 = _tc_update(proj[:h], closest1, W_upd, b_upd2)
    closest2 = _sc_gather(memory, idx2)
    upd2 = _tc_update(proj[h:], closest2, W_upd, b_upd2)
    idx = jnp.concatenate([idx1, idx2])
    upd = jnp.concatenate([upd1, upd2])
    closest = jnp.concatenate([closest1, closest2])
    updated = _sc_scatter_add(memory, idx, upd)
    return closest, updated


# speedup vs baseline: 1.0499x; 1.0499x over previous
import functools

import jax
import jax.numpy as jnp
from jax import lax
from jax.experimental import pallas as pl
from jax.experimental.pallas import tpu as pltpu
from jax.experimental.pallas import tpu_sc as plsc

N_TOK = 8192
DIM_IN = 768
DIM_H = 32
MEM_SIZE = 8192
MEM_RATE = 0.5

BLK = 512
CHUNK = 2048
NBLK = N_TOK // BLK


def _sc_geometry():
    try:
        info = plsc.get_sparse_core_info()
        return info.num_cores, info.num_subcores
    except Exception:
        return 2, 16


def _proj_body(z_ref, w_ref, b_ref, out_ref):
    out_ref[...] = (
        lax.dot_general(z_ref[...], w_ref[...], (((1,), (0,)), ((), ())))
        + b_ref[...]
    )


def _argmin_body(proj_ref, mem_ref, psum_ref, msum_ref, idx_ref):
    proj = proj_ref[...]
    prow = psum_ref[...]
    rowid = lax.broadcasted_iota(jnp.int32, (CHUNK, BLK), 0).astype(jnp.float32)

    def body(j, carry):
        minval, minidx = carry
        mem = mem_ref[pl.ds(j * CHUNK, CHUNK), :] * -2.0
        mcol = msum_ref[pl.ds(j * CHUNK, CHUNK), :]
        cross = lax.dot_general(mem, proj, (((1,), (1,)), ((), ())))
        d2 = (prow + mcol) + cross
        dist = jnp.where(d2 > 0.0, d2 * lax.rsqrt(d2), 0.0)
        cmin = jnp.min(dist, axis=0, keepdims=True)
        cidx = jnp.min(
            jnp.where(dist == cmin, rowid, jnp.float32(2**24)),
            axis=0, keepdims=True,
        ).astype(jnp.int32) + j * CHUNK
        take = cmin < minval
        newval = jnp.where(take, cmin, minval)
        newval = newval.astype(jnp.bfloat16).astype(jnp.float32)
        return (newval, jnp.where(take, cidx, minidx))

    init = (
        jnp.full((1, BLK), jnp.inf, jnp.float32),
        jnp.zeros((1, BLK), jnp.int32),
    )
    _, minidx = lax.fori_loop(0, MEM_SIZE // CHUNK, body, init)
    idx_ref[...] = minidx.reshape(1, 1, BLK)


def _update_body(proj_ref, closest_ref, w_ref, b_ref, out_ref):
    diff = proj_ref[...] - closest_ref[...]
    out_ref[...] = (
        lax.dot_general(diff, w_ref[...], (((1,), (0,)), ((), ())))
        + b_ref[...]
    ) * MEM_RATE


def _tc_proj(z, w, b2):
    return pl.pallas_call(
        _proj_body,
        grid=(NBLK,),
        in_specs=[
            pl.BlockSpec((BLK, DIM_IN), lambda i: (i, 0)),
            pl.BlockSpec((DIM_IN, DIM_H), lambda i: (0, 0)),
            pl.BlockSpec((1, DIM_H), lambda i: (0, 0)),
        ],
        out_specs=pl.BlockSpec((BLK, DIM_H), lambda i: (i, 0)),
        out_shape=jax.ShapeDtypeStruct((N_TOK, DIM_H), jnp.float32),
    )(z, w, b2)


def _tc_argmin(proj, memory, psum_row, msum_col):
    n_tok = proj.shape[0]
    nblk = n_tok // BLK
    idx3 = pl.pallas_call(
        _argmin_body,
        grid=(nblk,),
        in_specs=[
            pl.BlockSpec((BLK, DIM_H), lambda i: (i, 0)),
            pl.BlockSpec((MEM_SIZE, DIM_H), lambda i: (0, 0)),
            pl.BlockSpec((1, BLK), lambda i: (0, i)),
            pl.BlockSpec((MEM_SIZE, 1), lambda i: (0, 0)),
        ],
        out_specs=pl.BlockSpec((1, 1, BLK), lambda i: (i, 0, 0)),
        out_shape=jax.ShapeDtypeStruct((nblk, 1, BLK), jnp.int32),
    )(proj, memory, psum_row, msum_col)
    return idx3.reshape(n_tok)


def _tc_update(proj, closest, w_upd, b2):
    return pl.pallas_call(
        _update_body,
        grid=(proj.shape[0] // BLK,),
        in_specs=[
            pl.BlockSpec((BLK, DIM_H), lambda i: (i, 0)),
            pl.BlockSpec((BLK, DIM_H), lambda i: (i, 0)),
            pl.BlockSpec((DIM_H, DIM_H), lambda i: (0, 0)),
            pl.BlockSpec((1, DIM_H), lambda i: (0, 0)),
        ],
        out_specs=pl.BlockSpec((BLK, DIM_H), lambda i: (i, 0)),
        out_shape=jax.ShapeDtypeStruct((proj.shape[0], DIM_H), jnp.float32),
    )(proj, closest, w_upd, b2)


def _sc_gather(memory, idx):
    nc, ns = _sc_geometry()
    nw = nc * ns
    n_tok = idx.shape[0]
    bpw = n_tok // nw
    mesh = plsc.VectorSubcoreMesh(core_axis_name="c", subcore_axis_name="s")

    @functools.partial(
        pl.kernel,
        mesh=mesh,
        out_type=jax.ShapeDtypeStruct((n_tok, DIM_H), jnp.float32),
        scratch_types=[
            pltpu.VMEM((bpw,), jnp.int32),
            pltpu.VMEM((bpw, DIM_H), jnp.float32),
            pltpu.SemaphoreType.DMA,
        ],
        compiler_params=pltpu.CompilerParams(use_tc_tiling_on_sc=False),
    )
    def k(table_hbm, idx_hbm, out_hbm, idx_v, rows_v, sem):
        wid = lax.axis_index("s") * nc + lax.axis_index("c")
        base = wid * bpw
        pltpu.sync_copy(idx_hbm.at[pl.ds(base, bpw)], idx_v)
        pltpu.async_copy(table_hbm.at[idx_v], rows_v, sem).wait()
        pltpu.sync_copy(rows_v, out_hbm.at[pl.ds(base, bpw)])

    return k(memory, idx)


def _sc_scatter_add(memory, idx, upd):
    nc, ns = _sc_geometry()
    tok_pw = N_TOK // ns
    rows_init = MEM_SIZE // ns
    rows_out = MEM_SIZE // (nc * ns)
    mesh = plsc.VectorSubcoreMesh(core_axis_name="c", subcore_axis_name="s")

    @functools.partial(
        pl.kernel,
        mesh=mesh,
        out_type=jax.ShapeDtypeStruct((MEM_SIZE, DIM_H), jnp.float32),
        scratch_types=[
            pltpu.VMEM((tok_pw,), jnp.int32),
            pltpu.VMEM((tok_pw, DIM_H), jnp.float32),
            pltpu.VMEM_SHARED((MEM_SIZE, DIM_H), jnp.float32),
        ],
        compiler_params=pltpu.CompilerParams(use_tc_tiling_on_sc=False),
    )
    def k(mem_hbm, idx_hbm, upd_hbm, out_hbm, idx_v, upd_v, shared):
        cid = lax.axis_index("c")
        sid = lax.axis_index("s")
        pltpu.sync_copy(
            mem_hbm.at[pl.ds(sid * rows_init, rows_init)],
            shared.at[pl.ds(sid * rows_init, rows_init)],
        )
        plsc.subcore_barrier()
        base = sid * tok_pw
        pltpu.sync_copy(idx_hbm.at[pl.ds(base, tok_pw)], idx_v)
        pltpu.sync_copy(upd_hbm.at[pl.ds(base, tok_pw)], upd_v)
        pltpu.sync_copy(upd_v, shared.at[idx_v], add=True)
        plsc.subcore_barrier()
        wbase = cid * (MEM_SIZE // nc) + sid * rows_out
        pltpu.sync_copy(
            shared.at[pl.ds(wbase, rows_out)],
            out_hbm.at[pl.ds(wbase, rows_out)],
        )

    return k(memory, idx, upd)


def kernel(visual_embeddings, memory, W_proj, b_proj, W_upd, b_upd):
    b_proj2 = b_proj.reshape(1, DIM_H)
    b_upd2 = b_upd.reshape(1, DIM_H)

    proj = _tc_proj(visual_embeddings, W_proj, b_proj2)
    psum_row = jnp.sum(proj * proj, axis=1)[None, :]
    msum_col = jnp.sum(memory * memory, axis=1)[:, None]

    idx = _tc_argmin(proj, memory, psum_row, msum_col)
    closest = _sc_gather(memory, idx)
    upd = _tc_update(proj, closest, W_upd, b_upd2)
    updated = _sc_scatter_add(memory, idx, upd)
    return closest, updated
